# baseline (speedup 1.0000x reference)
import jax
import jax.numpy as jnp
from jax.experimental import pallas as pl
from jax.experimental.pallas import tpu as pltpu

_LANES = 128


def _body(B):
    def body(mask_hbm, hs_ref, out_ref, mask_v, copy_sem, row_sem):
        for b in range(B):
            pltpu.make_async_copy(
                mask_hbm.at[b], mask_v.at[b], copy_sem.at[b]
            ).start()
        for b in range(B):
            pltpu.make_async_copy(
                mask_hbm.at[b], mask_v.at[b], copy_sem.at[b]
            ).wait()
            total = jnp.sum(mask_v[b])
            idx = jnp.maximum(total - 1, 0)
            pltpu.make_async_copy(
                hs_ref.at[b, idx], out_ref.at[b], row_sem.at[b]
            ).start()
        for b in range(B):
            pltpu.make_async_copy(
                hs_ref.at[0, 0], out_ref.at[b], row_sem.at[b]
            ).wait()

    return body


def kernel(hidden_state, attention_mask):
    B, S, D = hidden_state.shape
    mask3 = attention_mask.reshape(B, S // _LANES, _LANES)
    return pl.pallas_call(
        _body(B),
        out_shape=jax.ShapeDtypeStruct((B, D), jnp.float32),
        in_specs=[
            pl.BlockSpec(memory_space=pltpu.MemorySpace.HBM),
            pl.BlockSpec(memory_space=pltpu.MemorySpace.HBM),
        ],
        out_specs=pl.BlockSpec(memory_space=pltpu.MemorySpace.HBM),
        scratch_shapes=[
            pltpu.VMEM((B, S // _LANES, _LANES), jnp.int32),
            pltpu.SemaphoreType.DMA((B,)),
            pltpu.SemaphoreType.DMA((B,)),
        ],
    )(mask3, hidden_state)


# speedup vs baseline: 1.4833x; 1.4833x over previous
import jax
import jax.numpy as jnp
from jax.experimental import pallas as pl
from jax.experimental.pallas import tpu as pltpu


def _body(B):
    def body(mask_ref, hs_ref, out_ref, sem):
        for b in range(B):
            total = jnp.sum(mask_ref[b, :])
            idx = jnp.maximum(total - 1, 0)
            pltpu.make_async_copy(
                hs_ref.at[b, idx], out_ref.at[b], sem.at[b]
            ).start()
        for b in range(B):
            pltpu.make_async_copy(
                hs_ref.at[0, 0], out_ref.at[b], sem.at[b]
            ).wait()

    return body


def kernel(hidden_state, attention_mask):
    B, S, D = hidden_state.shape
    return pl.pallas_call(
        _body(B),
        out_shape=jax.ShapeDtypeStruct((B, D), jnp.float32),
        in_specs=[
            pl.BlockSpec(memory_space=pltpu.VMEM),
            pl.BlockSpec(memory_space=pltpu.MemorySpace.HBM),
        ],
        out_specs=pl.BlockSpec(memory_space=pltpu.MemorySpace.HBM),
        scratch_shapes=[pltpu.SemaphoreType.DMA((B,))],
    )(attention_mask, hidden_state)
